# Initial kernel scaffold; baseline (speedup 1.0000x reference)
import jax
import jax.numpy as jnp
from jax.experimental import pallas as pl


def kernel(x, edge_index, W1, a_src1, a_dst1, W2, a_src2, a_dst2):
    raise NotImplementedError("write your pallas kernel here")


# speedup vs baseline: 6.7581x; 6.7581x over previous
import functools

import jax
import jax.numpy as jnp
from jax import lax
from jax.experimental import pallas as pl
from jax.experimental.pallas import tpu as pltpu
from jax.experimental.pallas import tpu_sc as plsc

N = 10000
D = 128
E = 320000
GAT_W = 0.8
ORG_W = 0.2
NEG_SLOPE = 0.2

NPAD = 10240
NC = 2
NS = 16
NW = NC * NS
EW = NPAD
EPAD = NW * EW
C = 64
NCHUNK = EW // C
NSUB = NPAD // NS

_f32 = jnp.float32
_i32 = jnp.int32


def _proj_body(x_ref, w_ref, a_ref, h_ref, s_ref):
    xb = x_ref[...]
    h = jnp.dot(xb, w_ref[...], preferred_element_type=_f32)
    h_ref[...] = h
    s_ref[...] = lax.dot_general(a_ref[...], h,
                                 dimension_numbers=(((1,), (1,)), ((), ())))


_BM = 512


def _proj(x, W, A):
    return pl.pallas_call(
        _proj_body,
        grid=(NPAD // _BM,),
        in_specs=[
            pl.BlockSpec((_BM, D), lambda i: (i, 0)),
            pl.BlockSpec((D, D), lambda i: (0, 0)),
            pl.BlockSpec((2, D), lambda i: (0, 0)),
        ],
        out_specs=[
            pl.BlockSpec((_BM, D), lambda i: (i, 0)),
            pl.BlockSpec((2, _BM), lambda i: (0, i)),
        ],
        out_shape=[
            jax.ShapeDtypeStruct((NPAD, D), _f32),
            jax.ShapeDtypeStruct((2, NPAD), _f32),
        ],
    )(x, W, A)


def _combine_body(agg_ref, den_ref, x_ref, o_ref):
    a = agg_ref[0] + agg_ref[1]
    dsum = den_ref[0, :] + den_ref[1, :] + 1e-16
    q = a / dsum[:, None]
    z = jnp.where(q > 0.0, q, jnp.exp(q) - 1.0)
    o_ref[...] = GAT_W * z + ORG_W * x_ref[...]


def _combine(agg, den, x):
    return pl.pallas_call(
        _combine_body,
        grid=(NPAD // _BM,),
        in_specs=[
            pl.BlockSpec((2, _BM, D), lambda i: (0, i, 0)),
            pl.BlockSpec((2, _BM), lambda i: (0, i)),
            pl.BlockSpec((_BM, D), lambda i: (i, 0)),
        ],
        out_specs=pl.BlockSpec((_BM, D), lambda i: (i, 0)),
        out_shape=jax.ShapeDtypeStruct((NPAD, D), _f32),
    )(agg, den, x)


def _sc_gat_body(h_hbm, s2_hbm, src_hbm, dst_hbm,
                 agg_out, den_out,
                 ssrc_v, sdst_v, srcb, dstb, eeb, rows,
                 shared_agg, shared_den, sem_g):
    c = lax.axis_index("c")
    s = lax.axis_index("s")
    wid = c * NS + s

    pltpu.sync_copy(s2_hbm.at[0], ssrc_v)
    pltpu.sync_copy(s2_hbm.at[1], sdst_v)

    def _zero_rows(i, _):
        for g in range(8):
            rows[0, i, pl.ds(g * 16, 16)] = jnp.zeros((16,), _f32)
        return 0
    lax.fori_loop(0, C, _zero_rows, 0)
    for g in range(C // 16):
        eeb[pl.ds(g * 16, 16)] = jnp.zeros((16,), _f32)

    for b in range(NSUB // C):
        pltpu.sync_copy(rows.at[0], shared_agg.at[pl.ds(s * NSUB + b * C, C)])
        pltpu.sync_copy(eeb, shared_den.at[pl.ds(s * NSUB + b * C, C)])

    plsc.subcore_barrier()

    def _chunk(j, _):
        b = lax.rem(j, 2)
        pltpu.sync_copy(src_hbm.at[wid, j], srcb.at[b])
        pltpu.sync_copy(dst_hbm.at[wid, j], dstb.at[b])
        cp = pltpu.async_copy(h_hbm.at[srcb.at[b]], rows.at[b], sem_g)

        for g in range(C // 16):
            si = srcb[b, pl.ds(g * 16, 16)]
            di = dstb[b, pl.ds(g * 16, 16)]
            ss = plsc.load_gather(ssrc_v, [si])
            sd = plsc.load_gather(sdst_v, [di])
            e = ss + sd
            e = jnp.where(e >= 0.0, e, e * NEG_SLOPE)
            eeb[pl.ds(g * 16, 16)] = jnp.exp(e)

        pltpu.sync_copy(eeb, shared_den.at[dstb.at[b]], add=True)

        cp.wait()

        def _scale(r, _):
            eb = plsc.load_gather(eeb, [jnp.full((16,), r, _i32)])
            for g in range(8):
                rows[b, r, pl.ds(g * 16, 16)] = (
                    rows[b, r, pl.ds(g * 16, 16)] * eb)
            return 0
        lax.fori_loop(0, C, _scale, 0)

        pltpu.sync_copy(rows.at[b], shared_agg.at[dstb.at[b]], add=True)
        return 0
    lax.fori_loop(0, NCHUNK, _chunk, 0)

    plsc.subcore_barrier()

    for b in range(NSUB // C):
        base = s * NSUB + b * C
        pltpu.sync_copy(shared_agg.at[pl.ds(base, C)], rows.at[0])
        pltpu.sync_copy(rows.at[0], agg_out.at[c, pl.ds(base, C)])
        pltpu.sync_copy(shared_den.at[pl.ds(base, C)], eeb)
        pltpu.sync_copy(eeb, den_out.at[c, pl.ds(base, C)])


_sc_gat = functools.partial(
    pl.kernel,
    mesh=plsc.VectorSubcoreMesh(core_axis_name="c", subcore_axis_name="s"),
    compiler_params=pltpu.CompilerParams(needs_layout_passes=False),
    out_type=[
        jax.ShapeDtypeStruct((NC, NPAD, D), _f32),
        jax.ShapeDtypeStruct((NC, NPAD), _f32),
    ],
    scratch_types=[
        pltpu.VMEM((NPAD,), _f32),
        pltpu.VMEM((NPAD,), _f32),
        pltpu.VMEM((2, C), _i32),
        pltpu.VMEM((2, C), _i32),
        pltpu.VMEM((C,), _f32),
        pltpu.VMEM((2, C, D), _f32),
        pltpu.VMEM_SHARED((NPAD, D), _f32),
        pltpu.VMEM_SHARED((NPAD,), _f32),
        pltpu.SemaphoreType.DMA,
    ],
)(_sc_gat_body)


def _gat_layer_fast(x, W, a_src, a_dst, src_r, dst_r):
    A = jnp.stack([a_src, a_dst])
    h, s2 = _proj(x, W, A)
    agg, den = _sc_gat(h, s2, src_r, dst_r)
    return _combine(agg, den, x)


def kernel(x, edge_index, W1, a_src1, a_dst1, W2, a_src2, a_dst2):
    xpad = jnp.pad(x, ((0, NPAD - N), (0, 0)))
    src = edge_index[0]
    dst = edge_index[1]
    src_r = jnp.concatenate(
        [src, jnp.zeros((EPAD - E,), _i32)]).reshape(NW, NCHUNK, C)
    dst_r = jnp.concatenate(
        [dst, jnp.full((EPAD - E,), NPAD - 1, _i32)]).reshape(NW, NCHUNK, C)

    h1 = _gat_layer_fast(xpad, W1, a_src1, a_dst1, src_r, dst_r)
    h2 = _gat_layer_fast(h1, W2, a_src2, a_dst2, src_r, dst_r)
    return h2[:N]
